# baseline (speedup 1.0000x reference)
import jax
import jax.numpy as jnp
from jax.experimental import pallas as pl
from jax.experimental.pallas import tpu as pltpu

DIM_OUT = 1000
KBLK = 40
NBUF = 16
KCH = DIM_OUT // KBLK


def _onehot_kernel(x_ref, o_hbm, stage, sem):
    C = x_ref.shape[0]
    B = x_ref.shape[2]
    nchunks = C * KCH

    def chunk_copy(i, b):
        c = jax.lax.div(i, KCH)
        k = jax.lax.rem(i, KCH)
        return pltpu.make_async_copy(
            stage.at[b],
            o_hbm.at[pl.ds(c, 1), pl.ds(k * KBLK, KBLK), :],
            sem.at[b],
        )

    iota = jax.lax.broadcasted_iota(jnp.int32, (1, KBLK, 1), 1)

    def body(i, _):
        b = jax.lax.rem(i, NBUF)

        @pl.when(i >= NBUF)
        def _():
            chunk_copy(i - NBUF, b).wait()

        c = jax.lax.div(i, KCH)
        k = jax.lax.rem(i, KCH)
        idx = x_ref[pl.ds(c, 1), :, :]
        stage.at[b][...] = (idx == iota + k * KBLK).astype(jnp.float32)
        chunk_copy(i, b).start()
        return 0

    jax.lax.fori_loop(0, nchunks, body, 0)

    def drain(i, _):
        chunk_copy(i, jax.lax.rem(i, NBUF)).wait()
        return 0

    jax.lax.fori_loop(nchunks - NBUF, nchunks, drain, 0)


def kernel(x):
    x = x.astype(jnp.int32)
    B, C = x.shape
    xt = x.T.reshape(C, 1, B)
    out_t = pl.pallas_call(
        _onehot_kernel,
        in_specs=[pl.BlockSpec(memory_space=pltpu.VMEM)],
        out_specs=pl.BlockSpec(memory_space=pl.ANY),
        out_shape=jax.ShapeDtypeStruct((C, DIM_OUT, B), jnp.float32),
        scratch_shapes=[
            pltpu.VMEM((NBUF, 1, KBLK, B), jnp.float32),
            pltpu.SemaphoreType.DMA((NBUF,)),
        ],
    )(xt)
    return jnp.transpose(out_t, (2, 0, 1))


# speedup vs baseline: 1.0157x; 1.0157x over previous
import jax
import jax.numpy as jnp
from jax.experimental import pallas as pl
from jax.experimental.pallas import tpu as pltpu

DIM_OUT = 1000
KBLK = 200
NBUF = 12
KCH = DIM_OUT // KBLK


def _onehot_kernel(x_ref, o_hbm, stage, sem):
    C = x_ref.shape[0]
    B = x_ref.shape[2]
    nchunks = C * KCH

    def chunk_copy(i, b):
        c = jax.lax.div(i, KCH)
        k = jax.lax.rem(i, KCH)
        return pltpu.make_async_copy(
            stage.at[b],
            o_hbm.at[pl.ds(c, 1), pl.ds(k * KBLK, KBLK), :],
            sem.at[b],
        )

    iota = jax.lax.broadcasted_iota(jnp.int32, (1, KBLK, 1), 1)

    def body(i, _):
        b = jax.lax.rem(i, NBUF)

        @pl.when(i >= NBUF)
        def _():
            chunk_copy(i - NBUF, b).wait()

        c = jax.lax.div(i, KCH)
        k = jax.lax.rem(i, KCH)
        idx = x_ref[pl.ds(c, 1), :, :]
        stage.at[b][...] = (idx == iota + k * KBLK).astype(jnp.float32)
        chunk_copy(i, b).start()
        return 0

    jax.lax.fori_loop(0, nchunks, body, 0)

    def drain(i, _):
        chunk_copy(i, jax.lax.rem(i, NBUF)).wait()
        return 0

    jax.lax.fori_loop(nchunks - NBUF, nchunks, drain, 0)


def kernel(x):
    x = x.astype(jnp.int32)
    B, C = x.shape
    xt = x.T.reshape(C, 1, B)
    out_t = pl.pallas_call(
        _onehot_kernel,
        in_specs=[pl.BlockSpec(memory_space=pltpu.VMEM)],
        out_specs=pl.BlockSpec(memory_space=pl.ANY),
        out_shape=jax.ShapeDtypeStruct((C, DIM_OUT, B), jnp.float32),
        scratch_shapes=[
            pltpu.VMEM((NBUF, 1, KBLK, B), jnp.float32),
            pltpu.SemaphoreType.DMA((NBUF,)),
        ],
    )(xt)
    return jnp.transpose(out_t, (2, 0, 1))
